# baseline (speedup 1.0000x reference)
import dataclasses
import functools

import jax
import jax.numpy as jnp
from jax import lax
from jax.experimental import pallas as pl
from jax.experimental.pallas import tpu as pltpu
from jax.experimental.pallas import tpu_sc as plsc

N = 10000
E = 160000
D = 256
DH = 128
NPAD = 10240
NC = 2
NS = 16
CH = 128
EP = 163840
NROW = EP // CH
RPW = NROW // NS
NPAIR = RPW // 2
ZR = 128
RPS = NPAD // NS


_RB = 400
_NRB = N // _RB


def _proj_body(h_ref, w2_ref, out_ref):
    out_ref[0] = lax.dot_general(
        h_ref[...], w2_ref[...],
        (((1,), (1,)), ((), ())),
        preferred_element_type=jnp.float32,
    )


def _proj(h, W2):
    return pl.pallas_call(
        _proj_body,
        grid=(_NRB, NC),
        in_specs=[
            pl.BlockSpec((_RB, D), lambda i, j: (i, 0)),
            pl.BlockSpec((DH, D), lambda i, j: (j, 0)),
        ],
        out_specs=pl.BlockSpec((1, _RB, DH), lambda i, j: (j, i, 0)),
        out_shape=jax.ShapeDtypeStruct((NC, NPAD, DH), jnp.float32),
    )(h, W2)


def _sc_agg(p2, ei2):
    mesh = plsc.VectorSubcoreMesh(core_axis_name="c", subcore_axis_name="s")
    cp = pltpu.CompilerParams()
    if "needs_layout_passes" in pltpu.CompilerParams.__dataclass_fields__:
        cp = dataclasses.replace(cp, needs_layout_passes=False)

    @functools.partial(
        pl.kernel,
        compiler_params=cp,
        out_type=(
            jax.ShapeDtypeStruct((NC * NPAD, DH), jnp.float32),
            jax.ShapeDtypeStruct((NC * NS, NPAD), jnp.float32),
        ),
        mesh=mesh,
        scratch_types=[
            pltpu.VMEM((2, 2, 2, CH), jnp.int32),
            pltpu.VMEM((2, CH, DH), jnp.float32),
            pltpu.VMEM((NPAD,), jnp.float32),
            pltpu.VMEM_SHARED((NPAD, DH), jnp.float32),
            pltpu.SemaphoreType.DMA,
            pltpu.SemaphoreType.DMA,
            pltpu.SemaphoreType.DMA,
            pltpu.SemaphoreType.DMA,
        ],
    )
    def k(p2_h, ei_h, acc_h, deg_h, ibuf, rows, hist, acc_sh,
          sg0, sg1, si0, si1):
        c = lax.axis_index("c")
        s = lax.axis_index("s")
        off = c * NPAD
        sg = (sg0, sg1)
        si = (si0, si1)

        zv = jnp.zeros((16,), jnp.float32)
        ov = jnp.ones((16,), jnp.float32)

        @pl.loop(0, ZR)
        def _(i):
            @pl.loop(0, DH, step=16)
            def _(j):
                rows[0, i, pl.ds(j, 16)] = zv

        @pl.loop(0, NPAD, step=16)
        def _(i):
            hist[pl.ds(i, 16)] = zv

        rbase = s * RPS

        @pl.loop(0, RPS, step=ZR)
        def _(r):
            pltpu.sync_copy(rows.at[0], acc_sh.at[pl.ds(rbase + r, ZR)])

        plsc.subcore_barrier()

        row0 = s * RPW

        def idx_copy(j, p):
            return pltpu.make_async_copy(
                ei_h.at[pl.ds(row0 + 2 * j, 2)], ibuf.at[p], si[p])

        def offset_add(p):
            for b in range(2):
                @pl.loop(0, CH, step=16)
                def _(jj, b=b):
                    ibuf[p, b, 0, pl.ds(jj, 16)] = (
                        ibuf[p, b, 0, pl.ds(jj, 16)] + off)

        def fire_g(p, b):
            return pltpu.async_copy(
                p2_h.at[ibuf.at[p].at[b].at[0]], rows.at[b], sg[b])

        def wait_g(p, b):
            pltpu.make_async_copy(
                p2_h.at[ibuf.at[p].at[b].at[0]], rows.at[b], sg[b]).wait()

        def scatter(p, b):
            pltpu.sync_copy(rows.at[b], acc_sh.at[ibuf.at[p].at[b].at[1]],
                            add=True)

            @pl.when(c == b)
            def _():
                for j in range(0, CH, 16):
                    plsc.addupdate_scatter(
                        hist, [ibuf[p, b, 1, pl.ds(j, 16)]], ov)

        idx_copy(0, 0).start()
        idx_copy(0, 0).wait()
        offset_add(0)
        fire_g(0, 0)
        fire_g(0, 1)
        idx_copy(1, 1).start()

        def phase(j, p):
            np_ = 1 - p

            @pl.when(j + 1 < NPAIR)
            def _():
                idx_copy(j + 1, np_).wait()
                offset_add(np_)

            for b in range(2):
                wait_g(p, b)
                scatter(p, b)

                @pl.when(j + 1 < NPAIR)
                def _(b=b):
                    fire_g(np_, b)

            @pl.when(j + 2 < NPAIR)
            def _():
                idx_copy(j + 2, p).start()

        @pl.loop(0, NPAIR, step=2)
        def _(j):
            phase(j, 0)
            phase(j + 1, 1)

        plsc.subcore_barrier()

        pltpu.sync_copy(acc_sh.at[pl.ds(rbase, RPS)],
                        acc_h.at[pl.ds(off + rbase, RPS)])
        pltpu.sync_copy(hist, deg_h.at[c * NS + s])

    return k(p2, ei2)


def _combine_body(h_ref, w1_ref, b2_ref, acc0_ref, acc1_ref, deg_ref, out_ref):
    q = lax.dot_general(
        h_ref[...], w1_ref[...],
        (((1,), (1,)), ((), ())),
        preferred_element_type=jnp.float32,
    ) + b2_ref[...]
    deg = jnp.sum(deg_ref[0], axis=0)[:, None]
    inv = 1.0 / jnp.maximum(deg, 1.0)
    hn = jnp.concatenate([acc0_ref[0], acc1_ref[0]], axis=1) * inv
    t = q + hn
    ss = jnp.sum(t * t, axis=1, keepdims=True)
    out_ref[...] = t / jnp.maximum(jnp.sqrt(ss), 1e-12)


def _combine(h, W1, b2, acc, deg):
    return pl.pallas_call(
        _combine_body,
        grid=(_NRB,),
        in_specs=[
            pl.BlockSpec((_RB, D), lambda i: (i, 0)),
            pl.BlockSpec((D, D), lambda i: (0, 0)),
            pl.BlockSpec((1, D), lambda i: (0, 0)),
            pl.BlockSpec((1, _RB, DH), lambda i: (0, i, 0)),
            pl.BlockSpec((1, _RB, DH), lambda i: (1, i, 0)),
            pl.BlockSpec((1, NC * NS, _RB), lambda i: (i, 0, 0)),
        ],
        out_specs=pl.BlockSpec((_RB, D), lambda i: (i, 0)),
        out_shape=jax.ShapeDtypeStruct((N, D), jnp.float32),
    )(h, W1, b2, acc, acc, deg)


def kernel(h, edge_index, W1, W2, b2):
    src = edge_index[0]
    dst = edge_index[1]
    pad = EP - E
    src2 = jnp.concatenate([src, jnp.zeros((pad,), jnp.int32)]).reshape(NROW, CH)
    dst2 = jnp.concatenate(
        [dst, N + (jnp.arange(pad, dtype=jnp.int32) % 16)]).reshape(NROW, CH)
    ei2 = jnp.stack([src2, dst2], axis=1)
    p2 = _proj(h, W2)
    acc, deg = _sc_agg(p2.reshape(NC * NPAD, DH), ei2)
    deg3 = deg[:, :N].reshape(NC * NS, _NRB, _RB).transpose(1, 0, 2)
    out = _combine(h, W1, b2.reshape(1, D), acc.reshape(NC, NPAD, DH), deg3)
    return out


# speedup vs baseline: 1.6243x; 1.6243x over previous
import dataclasses
import functools

import jax
import jax.numpy as jnp
from jax import lax
from jax.experimental import pallas as pl
from jax.experimental.pallas import tpu as pltpu
from jax.experimental.pallas import tpu_sc as plsc

N = 10000
E = 160000
D = 256
DH = 128
NPAD = 10240
NC = 2
NS = 16
EPS = E // NS
CH = 128
NCH = EPS // CH
TAIL = EPS - NCH * CH
ZR = 128
RPS = NPAD // NS


_RB = 400
_NRB = N // _RB


def _proj_body(h_ref, w2_ref, out_ref):
    out_ref[0] = lax.dot_general(
        h_ref[...], w2_ref[...],
        (((1,), (1,)), ((), ())),
        preferred_element_type=jnp.float32,
    )


def _proj(h, W2):
    return pl.pallas_call(
        _proj_body,
        grid=(_NRB, NC),
        in_specs=[
            pl.BlockSpec((_RB, D), lambda i, j: (i, 0)),
            pl.BlockSpec((DH, D), lambda i, j: (j, 0)),
        ],
        out_specs=pl.BlockSpec((1, _RB, DH), lambda i, j: (j, i, 0)),
        out_shape=jax.ShapeDtypeStruct((NC, NPAD, DH), jnp.float32),
    )(h, W2)


def _sc_agg(p2, src, dst):
    mesh = plsc.VectorSubcoreMesh(core_axis_name="c", subcore_axis_name="s")
    cp = pltpu.CompilerParams()
    if "needs_layout_passes" in pltpu.CompilerParams.__dataclass_fields__:
        cp = dataclasses.replace(cp, needs_layout_passes=False)

    @functools.partial(
        pl.kernel,
        compiler_params=cp,
        out_type=(
            jax.ShapeDtypeStruct((NC * NPAD, DH), jnp.float32),
            jax.ShapeDtypeStruct((NS, NPAD), jnp.float32),
        ),
        mesh=mesh,
        scratch_types=[
            pltpu.VMEM((2, CH), jnp.int32),
            pltpu.VMEM((2, CH), jnp.int32),
            pltpu.VMEM((2, CH, DH), jnp.float32),
            pltpu.VMEM((TAIL,), jnp.int32),
            pltpu.VMEM((TAIL,), jnp.int32),
            pltpu.VMEM((TAIL, DH), jnp.float32),
            pltpu.VMEM((NPAD,), jnp.float32),
            pltpu.VMEM_SHARED((NPAD, DH), jnp.float32),
            pltpu.SemaphoreType.DMA,
            pltpu.SemaphoreType.DMA,
        ],
    )
    def k(p2_h, src_h, dst_h, acc_h, deg_h,
          isrc, idst, rows, tsrc, tdst, trows, hist,
          acc_sh, sg0, sg1):
        c = lax.axis_index("c")
        s = lax.axis_index("s")
        core0 = c == 0
        off = c * NPAD

        zv = jnp.zeros((16,), jnp.float32)
        ov = jnp.ones((16,), jnp.float32)

        @pl.loop(0, ZR)
        def _(i):
            @pl.loop(0, DH, step=16)
            def _(j):
                rows[0, i, pl.ds(j, 16)] = zv

        @pl.loop(0, NPAD, step=16)
        def _(i):
            hist[pl.ds(i, 16)] = zv

        rbase = s * RPS

        @pl.loop(0, RPS, step=ZR)
        def _(r):
            pltpu.sync_copy(rows.at[0], acc_sh.at[pl.ds(rbase + r, ZR)])

        plsc.subcore_barrier()

        ebase = s * EPS

        def load_idx(k_, b):
            pltpu.sync_copy(src_h.at[pl.ds(ebase + k_ * CH, CH)], isrc.at[b])
            pltpu.sync_copy(dst_h.at[pl.ds(ebase + k_ * CH, CH)], idst.at[b])

            @pl.loop(0, CH, step=16)
            def _(j):
                isrc[b, pl.ds(j, 16)] = isrc[b, pl.ds(j, 16)] + off

        def scatter(b):
            pltpu.sync_copy(rows.at[b], acc_sh.at[idst.at[b]], add=True)

            @pl.when(core0)
            def _():
                @pl.loop(0, CH, step=16)
                def _(j):
                    plsc.addupdate_scatter(hist, [idst[b, pl.ds(j, 16)]], ov)

        @pl.loop(0, NCH, step=2)
        def _(k_):
            load_idx(k_, 0)
            d0 = pltpu.async_copy(p2_h.at[isrc.at[0]], rows.at[0], sg0)
            load_idx(k_ + 1, 1)
            d1 = pltpu.async_copy(p2_h.at[isrc.at[1]], rows.at[1], sg1)
            d0.wait()
            scatter(0)
            d1.wait()
            scatter(1)

        tbase = ebase + NCH * CH
        pltpu.sync_copy(src_h.at[pl.ds(tbase, TAIL)], tsrc)
        pltpu.sync_copy(dst_h.at[pl.ds(tbase, TAIL)], tdst)
        tsrc[...] = tsrc[...] + off
        pltpu.sync_copy(p2_h.at[tsrc], trows)
        pltpu.sync_copy(trows, acc_sh.at[tdst], add=True)

        @pl.when(core0)
        def _():
            plsc.addupdate_scatter(hist, [tdst[...]], ov)

        plsc.subcore_barrier()

        pltpu.sync_copy(acc_sh.at[pl.ds(rbase, RPS)],
                        acc_h.at[pl.ds(off + rbase, RPS)])

        @pl.when(core0)
        def _():
            pltpu.sync_copy(hist, deg_h.at[s])

    return k(p2, src, dst)


def _combine_body(h_ref, w1_ref, b2_ref, acc0_ref, acc1_ref, deg_ref, out_ref):
    q = lax.dot_general(
        h_ref[...], w1_ref[...],
        (((1,), (1,)), ((), ())),
        preferred_element_type=jnp.float32,
    ) + b2_ref[...]
    deg = jnp.sum(deg_ref[0], axis=0)[:, None]
    inv = 1.0 / jnp.maximum(deg, 1.0)
    hn = jnp.concatenate([acc0_ref[0], acc1_ref[0]], axis=1) * inv
    t = q + hn
    ss = jnp.sum(t * t, axis=1, keepdims=True)
    out_ref[...] = t / jnp.maximum(jnp.sqrt(ss), 1e-12)


def _combine(h, W1, b2, acc, deg):
    return pl.pallas_call(
        _combine_body,
        grid=(_NRB,),
        in_specs=[
            pl.BlockSpec((_RB, D), lambda i: (i, 0)),
            pl.BlockSpec((D, D), lambda i: (0, 0)),
            pl.BlockSpec((1, D), lambda i: (0, 0)),
            pl.BlockSpec((1, _RB, DH), lambda i: (0, i, 0)),
            pl.BlockSpec((1, _RB, DH), lambda i: (1, i, 0)),
            pl.BlockSpec((1, NS, _RB), lambda i: (i, 0, 0)),
        ],
        out_specs=pl.BlockSpec((_RB, D), lambda i: (i, 0)),
        out_shape=jax.ShapeDtypeStruct((N, D), jnp.float32),
    )(h, W1, b2, acc, acc, deg)


def kernel(h, edge_index, W1, W2, b2):
    src = edge_index[0]
    dst = edge_index[1]
    p2 = _proj(h, W2)
    acc, deg = _sc_agg(p2.reshape(NC * NPAD, DH), src, dst)
    deg3 = deg[:, :N].reshape(NS, _NRB, _RB).transpose(1, 0, 2)
    out = _combine(h, W1, b2.reshape(1, D), acc.reshape(NC, NPAD, DH), deg3)
    return out
